# baseline (speedup 1.0000x reference)
import functools

import jax
import jax.numpy as jnp
from jax import lax
from jax.experimental import pallas as pl
from jax.experimental.pallas import tpu as pltpu
from jax.experimental.pallas import tpu_sc as plsc

_NC = 2
_NS = 16
_NW = _NC * _NS

_CHUNK = 256
_NBUF = 4


def _gather_grid(n, v, d):
    b_per_w = n // _NW
    n_chunks = b_per_w // _CHUNK
    n_groups = n_chunks // _NBUF
    stage = v // _NS

    mesh = plsc.VectorSubcoreMesh(core_axis_name="c", subcore_axis_name="s")

    scratch = (
        [pltpu.VMEM((_CHUNK,), jnp.int32) for _ in range(_NBUF)]
        + [pltpu.VMEM((_CHUNK, d), jnp.float32) for _ in range(_NBUF)]
        + [pltpu.SemaphoreType.DMA for _ in range(2 * _NBUF)]
        + [pltpu.VMEM_SHARED((v, d), jnp.float32)]
    )

    @functools.partial(
        pl.kernel,
        mesh=mesh,
        out_type=jax.ShapeDtypeStruct((n, d), jnp.float32),
        scratch_types=scratch,
        compiler_params=pltpu.CompilerParams(use_tc_tiling_on_sc=False),
    )
    def gather_k(idx_hbm, pe_hbm, out_hbm, *bufs):
        idx_v = bufs[:_NBUF]
        rows_v = bufs[_NBUF:2 * _NBUF]
        gsem = bufs[2 * _NBUF:3 * _NBUF]
        ssem = bufs[3 * _NBUF:4 * _NBUF]
        pe_sh = bufs[4 * _NBUF]

        sid = lax.axis_index("s")
        wid = sid * _NC + lax.axis_index("c")
        base = wid * b_per_w

        srow = pl.multiple_of(sid * stage, 8)
        stage_v = rows_v[0].at[pl.ds(0, stage)]
        pltpu.sync_copy(pe_hbm.at[pl.ds(srow, stage)], stage_v)
        pltpu.sync_copy(stage_v, pe_sh.at[pl.ds(srow, stage)])
        plsc.subcore_barrier()

        def chunk_off(j):
            return pl.multiple_of(base + j * _CHUNK, 8)

        def load_and_gather(j, b):
            pltpu.sync_copy(idx_hbm.at[pl.ds(chunk_off(j), _CHUNK)], idx_v[b])

        def wait_gather_start_store(j, b):
            pltpu.async_copy(rows_v[b],
                             out_hbm.at[pl.ds(chunk_off(j), _CHUNK)], ssem[b])

        def wait_store(j, b):
            pltpu.make_async_copy(rows_v[b],
                                  out_hbm.at[pl.ds(chunk_off(j), _CHUNK)],
                                  ssem[b]).wait()

        for b in range(_NBUF):
            load_and_gather(b, b)

        def body(g, carry):
            j0 = g * _NBUF
            for b in range(_NBUF):
                wait_gather_start_store(j0 + b, b)
            for b in range(_NBUF):
                wait_store(j0 + b, b)
                load_and_gather(j0 + b + _NBUF, b)
            return carry

        lax.fori_loop(0, n_groups - 1, body, 0)

        j0 = (n_groups - 1) * _NBUF
        for b in range(_NBUF):
            wait_gather_start_store(j0 + b, b)
        for b in range(_NBUF):
            wait_store(j0 + b, b)

    return gather_k


def kernel(positions, pe):
    b, s = positions.shape
    v, d = pe.shape
    n = b * s
    idx_flat = positions.reshape(n).astype(jnp.int32)
    out = _gather_grid(n, v, d)(idx_flat, pe)
    return out.reshape(b, s, d)


# speedup vs baseline: 1.0191x; 1.0191x over previous
import functools

import jax
import jax.numpy as jnp
from jax import lax
from jax.experimental import pallas as pl
from jax.experimental.pallas import tpu as pltpu
from jax.experimental.pallas import tpu_sc as plsc

_NC = 2
_NS = 16
_NW = _NC * _NS

_CHUNK = 256
_NBUF = 4


def _gather_grid(n, v, d):
    b_per_w = n // _NW
    n_chunks = b_per_w // _CHUNK
    n_groups = n_chunks // _NBUF
    stage = v // _NS

    mesh = plsc.VectorSubcoreMesh(core_axis_name="c", subcore_axis_name="s")

    scratch = (
        [pltpu.VMEM((_CHUNK,), jnp.int32) for _ in range(_NBUF)]
        + [pltpu.VMEM((_CHUNK, d), jnp.float32) for _ in range(_NBUF)]
        + [pltpu.SemaphoreType.DMA for _ in range(2 * _NBUF)]
        + [pltpu.VMEM_SHARED((v, d), jnp.float32)]
    )

    @functools.partial(
        pl.kernel,
        mesh=mesh,
        out_type=jax.ShapeDtypeStruct((n, d), jnp.float32),
        scratch_types=scratch,
        compiler_params=pltpu.CompilerParams(use_tc_tiling_on_sc=False),
    )
    def gather_k(idx_hbm, pe_hbm, out_hbm, *bufs):
        idx_v = bufs[:_NBUF]
        rows_v = bufs[_NBUF:2 * _NBUF]
        gsem = bufs[2 * _NBUF:3 * _NBUF]
        ssem = bufs[3 * _NBUF:4 * _NBUF]
        pe_sh = bufs[4 * _NBUF]

        sid = lax.axis_index("s")
        wid = sid * _NC + lax.axis_index("c")
        base = wid * b_per_w

        srow = pl.multiple_of(sid * stage, 8)
        stage_v = rows_v[0].at[pl.ds(0, stage)]
        pltpu.sync_copy(pe_hbm.at[pl.ds(srow, stage)], stage_v)
        pltpu.sync_copy(stage_v, pe_sh.at[pl.ds(srow, stage)])
        plsc.subcore_barrier()

        def chunk_off(j):
            return pl.multiple_of(base + j * _CHUNK, 8)

        def load_and_gather(j, b):
            pltpu.sync_copy(idx_hbm.at[pl.ds(chunk_off(j), _CHUNK)], idx_v[b])
            pltpu.async_copy(pe_sh.at[idx_v[b]], rows_v[b], gsem[b])

        def wait_gather_start_store(j, b):
            pltpu.make_async_copy(pe_sh.at[idx_v[b]], rows_v[b],
                                  gsem[b]).wait()
            pltpu.async_copy(rows_v[b],
                             out_hbm.at[pl.ds(chunk_off(j), _CHUNK)], ssem[b])

        def wait_store(j, b):
            pltpu.make_async_copy(rows_v[b],
                                  out_hbm.at[pl.ds(chunk_off(j), _CHUNK)],
                                  ssem[b]).wait()

        for b in range(_NBUF):
            load_and_gather(b, b)

        def body(g, carry):
            j0 = g * _NBUF
            for b in range(_NBUF):
                wait_gather_start_store(j0 + b, b)
            for b in range(_NBUF):
                wait_store(j0 + b, b)
                load_and_gather(j0 + b + _NBUF, b)
            return carry

        lax.fori_loop(0, n_groups - 1, body, 0)

        j0 = (n_groups - 1) * _NBUF
        for b in range(_NBUF):
            wait_gather_start_store(j0 + b, b)
        for b in range(_NBUF):
            wait_store(j0 + b, b)

    return gather_k


def kernel(positions, pe):
    b, s = positions.shape
    v, d = pe.shape
    n = b * s
    idx_flat = positions.reshape(n).astype(jnp.int32)
    out = _gather_grid(n, v, d)(idx_flat, pe)
    return out.reshape(b, s, d)
